# baseline (speedup 1.0000x reference)
import functools

import jax
import jax.numpy as jnp
from jax import lax
from jax.experimental import pallas as pl
from jax.experimental.pallas import tpu as pltpu
from jax.experimental.pallas import tpu_sc as plsc

TOPK = 16
GROUP = 128
_NEG = float("-inf")
_BIG = 2**30
_BIGF = float(2**24)


def _stage1_body(q_ref, k_ref, imp_ref, s_ref, gsel_ref, m_ref, *, K):
    ik = pl.program_id(1)
    nk = pl.num_programs(1)
    qt, kt = s_ref.shape
    ng_t = kt // GROUP

    q = q_ref[...]
    kk = k_ref[...]
    imp = imp_ref[...]
    qn = q / (jnp.sqrt(jnp.sum(q * q, axis=1, keepdims=True)) + 1e-8)
    kn = kk / (jnp.sqrt(jnp.sum(kk * kk, axis=1, keepdims=True)) + 1e-8)
    s = lax.dot_general(qn.astype(jnp.bfloat16), kn.astype(jnp.bfloat16),
                        (((1,), (1,)), ((), ())),
                        preferred_element_type=jnp.float32)
    s = s * imp

    def _mask(x):
        col = ik * kt + lax.broadcasted_iota(jnp.int32, x.shape, 1)
        return jnp.where(col < K, x, _NEG)

    s = lax.cond(ik == nk - 1, _mask, lambda x: x, s)
    s_ref[...] = s

    g = s.reshape(qt, ng_t, GROUP)
    m_ref[ik] = jnp.max(g, axis=2)

    @pl.when(ik == nk - 1)
    def _():
        w0 = jnp.concatenate([m_ref[i] for i in range(nk)], axis=1)
        gid = lax.broadcasted_iota(jnp.int32, w0.shape, 1).astype(jnp.float32)
        lane = lax.broadcasted_iota(jnp.int32, (qt, TOPK), 1)

        def body(t, carry):
            w, acc = carry
            mx = jnp.max(w, axis=1, keepdims=True)
            sel = jnp.min(jnp.where(w == mx, gid, _BIGF), axis=1,
                          keepdims=True)
            w = jnp.where(gid == sel, _NEG, w)
            acc = jnp.where(lane == t, sel, acc)
            return w, acc

        _, acc = lax.fori_loop(0, TOPK, body,
                               (w0, jnp.zeros((qt, TOPK), jnp.float32)))
        gsel_ref[...] = acc.astype(jnp.int32)


def _stage1(queries, keys, importance, K, Qt, Kt):
    Q, D = queries.shape
    Kp = keys.shape[0]
    nq, nk = Q // Qt, Kp // Kt
    imp2 = importance.reshape(1, Kp)
    return pl.pallas_call(
        functools.partial(_stage1_body, K=K),
        grid=(nq, nk),
        in_specs=[
            pl.BlockSpec((Qt, D), lambda iq, ik: (iq, 0)),
            pl.BlockSpec((Kt, D), lambda iq, ik: (ik, 0)),
            pl.BlockSpec((1, Kt), lambda iq, ik: (0, ik)),
        ],
        out_specs=[
            pl.BlockSpec((Qt, Kt), lambda iq, ik: (iq, ik)),
            pl.BlockSpec((Qt, TOPK), lambda iq, ik: (iq, 0)),
        ],
        out_shape=[
            jax.ShapeDtypeStruct((Q, Kp), jnp.float32),
            jax.ShapeDtypeStruct((Q, TOPK), jnp.int32),
        ],
        scratch_shapes=[pltpu.VMEM((nk, Qt, Kt // GROUP), jnp.float32)],
    )(queries, keys, imp2)


def _stage2_sc(s2, gsel):
    Q = gsel.shape[0]
    ngroups = s2.shape[0] // Q
    NW = 32
    QPW = Q // NW
    CH = 8
    mesh = plsc.VectorSubcoreMesh(core_axis_name="c", subcore_axis_name="s")

    @functools.partial(
        pl.kernel, mesh=mesh,
        out_type=jax.ShapeDtypeStruct((Q * TOPK, GROUP), jnp.float32),
        scratch_types=[
            pltpu.VMEM((QPW, TOPK), jnp.int32),
            pltpu.VMEM((CH * TOPK,), jnp.int32),
            pltpu.VMEM((CH * TOPK, GROUP), jnp.float32),
            pltpu.SemaphoreType.DMA,
        ],
    )
    def k(s2_hbm, gsel_hbm, out_hbm, gsel_v, idx_v, cand_v, sem):
        wid = lax.axis_index("s") * 2 + lax.axis_index("c")
        q0 = wid * QPW
        pltpu.sync_copy(gsel_hbm.at[pl.ds(q0, QPW)], gsel_v)

        def chunk(t, carry):
            qc = q0 + t * CH
            for i in range(CH):
                g = gsel_v[t * CH + i, :]
                idx_v[pl.ds(i * TOPK, TOPK)] = g + (qc + i) * ngroups
            pltpu.async_copy(s2_hbm.at[idx_v], cand_v, sem).wait()
            pltpu.sync_copy(cand_v, out_hbm.at[pl.ds(qc * TOPK, CH * TOPK)])
            return carry

        lax.fori_loop(0, QPW // CH, chunk, 0)

    return k(s2, gsel)


def _stage3_body(cand_ref, gsel_ref, vals_ref, idx_ref):
    qt = cand_ref.shape[0]
    c = cand_ref[...]
    gsel = gsel_ref[...]
    gidx3 = (gsel.astype(jnp.float32) * GROUP
             + lax.broadcasted_iota(
                 jnp.int32, (qt, TOPK, GROUP), 2).astype(jnp.float32))
    gidx = gidx3.reshape(qt, TOPK * GROUP)
    lane = lax.broadcasted_iota(jnp.int32, (qt, TOPK), 1)

    def body(t, carry):
        w, vacc, iacc = carry
        mx = jnp.max(w, axis=1, keepdims=True)
        sel = jnp.min(jnp.where(w == mx, gidx, _BIGF), axis=1, keepdims=True)
        w = jnp.where(gidx == sel, _NEG, w)
        vacc = jnp.where(lane == t, mx, vacc)
        iacc = jnp.where(lane == t, sel, iacc)
        return w, vacc, iacc

    _, vacc, iacc = lax.fori_loop(
        0, TOPK, body, (c, jnp.zeros((qt, TOPK), jnp.float32),
                        jnp.zeros((qt, TOPK), jnp.float32)))
    vals_ref[...] = vacc
    idx_ref[...] = iacc.astype(jnp.int32)


def _stage3(cand2, gsel3, Qt):
    Q = cand2.shape[0]
    return pl.pallas_call(
        _stage3_body,
        grid=(Q // Qt,),
        in_specs=[
            pl.BlockSpec((Qt, TOPK * GROUP), lambda iq: (iq, 0)),
            pl.BlockSpec((Qt, TOPK, 1), lambda iq: (iq, 0, 0)),
        ],
        out_specs=[
            pl.BlockSpec((Qt, TOPK), lambda iq: (iq, 0)),
            pl.BlockSpec((Qt, TOPK), lambda iq: (iq, 0)),
        ],
        out_shape=[
            jax.ShapeDtypeStruct((Q, TOPK), jnp.float32),
            jax.ShapeDtypeStruct((Q, TOPK), jnp.int32),
        ],
    )(cand2, gsel3)


def kernel(queries, keys, importance, k):
    Q, D = queries.shape
    K = keys.shape[0]
    Qt = min(512, Q)
    Kt = 4096
    Kp = -(-K // Kt) * Kt
    if Kp != K:
        keys = jnp.pad(keys, ((0, Kp - K), (0, 0)))
        importance = jnp.pad(importance, (0, Kp - K))
    s, gsel = _stage1(queries, keys, importance, K, Qt, Kt)
    cand = _stage2_sc(s.reshape(Q * (Kp // GROUP), GROUP), gsel)
    vals, idx = _stage3(cand.reshape(Q, TOPK * GROUP),
                        gsel.reshape(Q, TOPK, 1), min(1024, Q))
    return vals, idx


# speedup vs baseline: 1.0574x; 1.0574x over previous
import functools

import jax
import jax.numpy as jnp
from jax import lax
from jax.experimental import pallas as pl
from jax.experimental.pallas import tpu as pltpu
from jax.experimental.pallas import tpu_sc as plsc

TOPK = 16
GROUP = 128
_NEG = float("-inf")
_BIG = 2**30
_BIGF = float(2**24)


def _stage1_body(q_ref, k_ref, imp_ref, s_ref, gsel_ref, m_ref, *, K):
    ik = pl.program_id(1)
    nk = pl.num_programs(1)
    qt, kt = s_ref.shape
    ng_t = kt // GROUP

    q = q_ref[...]
    kk = k_ref[...]
    imp = imp_ref[...]
    qn = q / (jnp.sqrt(jnp.sum(q * q, axis=1, keepdims=True)) + 1e-8)
    kn = kk / (jnp.sqrt(jnp.sum(kk * kk, axis=1, keepdims=True)) + 1e-8)
    s = lax.dot_general(qn.astype(jnp.bfloat16), kn.astype(jnp.bfloat16),
                        (((1,), (1,)), ((), ())),
                        preferred_element_type=jnp.float32)
    s = s * imp

    def _mask(x):
        col = ik * kt + lax.broadcasted_iota(jnp.int32, x.shape, 1)
        return jnp.where(col < K, x, _NEG)

    s = lax.cond(ik == nk - 1, _mask, lambda x: x, s)
    s_ref[...] = s

    g = s.reshape(qt, ng_t, GROUP)
    m_ref[ik] = jnp.max(g, axis=2)

    @pl.when(ik == nk - 1)
    def _():
        w = jnp.concatenate([m_ref[i] for i in range(nk)], axis=1)
        gid = lax.broadcasted_iota(jnp.int32, w.shape, 1).astype(jnp.float32)
        sel_cols = []
        for _t in range(TOPK):
            mx = jnp.max(w, axis=1, keepdims=True)
            sel = jnp.min(jnp.where(w == mx, gid, _BIGF), axis=1,
                          keepdims=True)
            w = jnp.where(gid == sel, _NEG, w)
            sel_cols.append(sel)
        gsel_ref[...] = jnp.concatenate(sel_cols, axis=1).astype(jnp.int32)


def _stage1(queries, keys, importance, K, Qt, Kt):
    Q, D = queries.shape
    Kp = keys.shape[0]
    nq, nk = Q // Qt, Kp // Kt
    imp2 = importance.reshape(1, Kp)
    return pl.pallas_call(
        functools.partial(_stage1_body, K=K),
        grid=(nq, nk),
        in_specs=[
            pl.BlockSpec((Qt, D), lambda iq, ik: (iq, 0)),
            pl.BlockSpec((Kt, D), lambda iq, ik: (ik, 0)),
            pl.BlockSpec((1, Kt), lambda iq, ik: (0, ik)),
        ],
        out_specs=[
            pl.BlockSpec((Qt, Kt), lambda iq, ik: (iq, ik)),
            pl.BlockSpec((Qt, TOPK), lambda iq, ik: (iq, 0)),
        ],
        out_shape=[
            jax.ShapeDtypeStruct((Q, Kp), jnp.float32),
            jax.ShapeDtypeStruct((Q, TOPK), jnp.int32),
        ],
        scratch_shapes=[pltpu.VMEM((nk, Qt, Kt // GROUP), jnp.float32)],
    )(queries, keys, imp2)


def _stage2_sc(s2, gsel):
    Q = gsel.shape[0]
    ngroups = s2.shape[0] // Q
    NW = 32
    QPW = Q // NW
    CH = 8
    mesh = plsc.VectorSubcoreMesh(core_axis_name="c", subcore_axis_name="s")

    @functools.partial(
        pl.kernel, mesh=mesh,
        out_type=jax.ShapeDtypeStruct((Q * TOPK, GROUP), jnp.float32),
        scratch_types=[
            pltpu.VMEM((QPW, TOPK), jnp.int32),
            pltpu.VMEM((CH * TOPK,), jnp.int32),
            pltpu.VMEM((CH * TOPK, GROUP), jnp.float32),
            pltpu.SemaphoreType.DMA,
        ],
    )
    def k(s2_hbm, gsel_hbm, out_hbm, gsel_v, idx_v, cand_v, sem):
        wid = lax.axis_index("s") * 2 + lax.axis_index("c")
        q0 = wid * QPW
        pltpu.sync_copy(gsel_hbm.at[pl.ds(q0, QPW)], gsel_v)

        def chunk(t, carry):
            qc = q0 + t * CH
            for i in range(CH):
                g = gsel_v[t * CH + i, :]
                idx_v[pl.ds(i * TOPK, TOPK)] = g + (qc + i) * ngroups
            pltpu.async_copy(s2_hbm.at[idx_v], cand_v, sem).wait()
            pltpu.sync_copy(cand_v, out_hbm.at[pl.ds(qc * TOPK, CH * TOPK)])
            return carry

        lax.fori_loop(0, QPW // CH, chunk, 0)

    return k(s2, gsel)


def _stage3_body(cand_ref, gsel_ref, vals_ref, idx_ref):
    qt = cand_ref.shape[0]
    c = cand_ref[...]
    gsel = gsel_ref[...]
    gidx3 = (gsel.astype(jnp.float32) * GROUP
             + lax.broadcasted_iota(
                 jnp.int32, (qt, TOPK, GROUP), 2).astype(jnp.float32))
    gidx = gidx3.reshape(qt, TOPK * GROUP)
    vals_cols, idx_cols = [], []
    w = c
    for _t in range(TOPK):
        mx = jnp.max(w, axis=1, keepdims=True)
        sel = jnp.min(jnp.where(w == mx, gidx, _BIGF), axis=1, keepdims=True)
        w = jnp.where(gidx == sel, _NEG, w)
        vals_cols.append(mx)
        idx_cols.append(sel)
    vals_ref[...] = jnp.concatenate(vals_cols, axis=1)
    idx_ref[...] = jnp.concatenate(idx_cols, axis=1).astype(jnp.int32)


def _stage3(cand2, gsel3, Qt):
    Q = cand2.shape[0]
    return pl.pallas_call(
        _stage3_body,
        grid=(Q // Qt,),
        in_specs=[
            pl.BlockSpec((Qt, TOPK * GROUP), lambda iq: (iq, 0)),
            pl.BlockSpec((Qt, TOPK, 1), lambda iq: (iq, 0, 0)),
        ],
        out_specs=[
            pl.BlockSpec((Qt, TOPK), lambda iq: (iq, 0)),
            pl.BlockSpec((Qt, TOPK), lambda iq: (iq, 0)),
        ],
        out_shape=[
            jax.ShapeDtypeStruct((Q, TOPK), jnp.float32),
            jax.ShapeDtypeStruct((Q, TOPK), jnp.int32),
        ],
    )(cand2, gsel3)


def kernel(queries, keys, importance, k):
    Q, D = queries.shape
    K = keys.shape[0]
    Qt = min(512, Q)
    Kt = 4096
    Kp = -(-K // Kt) * Kt
    if Kp != K:
        keys = jnp.pad(keys, ((0, Kp - K), (0, 0)))
        importance = jnp.pad(importance, (0, Kp - K))
    s, gsel = _stage1(queries, keys, importance, K, Qt, Kt)
    cand = _stage2_sc(s.reshape(Q * (Kp // GROUP), GROUP), gsel)
    vals, idx = _stage3(cand.reshape(Q, TOPK * GROUP),
                        gsel.reshape(Q, TOPK, 1), min(512, Q))
    return vals, idx
